# baseline (speedup 1.0000x reference)
import functools

import jax
import jax.numpy as jnp
from jax import lax
from jax.experimental import pallas as pl
from jax.experimental.pallas import tpu as pltpu
from jax.experimental.pallas import tpu_sc as plsc

DIM = 64
PAD = 128
NUM_WORKERS = 32
NBUF = 4


def _emb_body(rows_per_w, seq, idx_hbm, table_hbm, out_hbm, idx_v, rows_v,
              sem_in, sem_out):
    cid = lax.axis_index("c")
    sid = lax.axis_index("s")
    wid = sid * 2 + cid
    base = wid * rows_per_w
    pltpu.sync_copy(idx_hbm.at[pl.ds(base, rows_per_w)], idx_v)

    def fire_gather(r, buf):
        pltpu.async_copy(table_hbm.at[idx_v.at[r]], rows_v.at[buf], sem_in)

    def wait_one_gather():
        pltpu.make_async_copy(
            table_hbm.at[idx_v.at[0]], rows_v.at[0], sem_in).wait()

    def store(r, buf):
        pltpu.async_copy(
            rows_v.at[buf],
            out_hbm.at[pl.ds((base + r) * seq, seq), pl.ds(0, DIM)], sem_out)

    def wait_one_store():
        pltpu.make_async_copy(
            rows_v.at[0],
            out_hbm.at[pl.ds(0, seq), pl.ds(0, DIM)], sem_out).wait()

    for k in range(NBUF - 1):
        fire_gather(k, k)

    def body(r, carry):
        buf = lax.rem(r, NBUF)

        @pl.when(jnp.logical_and(r >= 1, r + NBUF - 1 < rows_per_w))
        def _():
            wait_one_store()

        @pl.when(r + NBUF - 1 < rows_per_w)
        def _():
            fire_gather(r + NBUF - 1, lax.rem(r + NBUF - 1, NBUF))

        wait_one_gather()
        store(r, buf)
        return carry

    lax.fori_loop(0, rows_per_w, body, 0)
    for _k in range(NBUF):
        wait_one_store()


def kernel(token_ids, weight):
    b, s = token_ids.shape
    v, d = weight.shape
    assert b % NUM_WORKERS == 0 and d == DIM
    rows_per_w = b // NUM_WORKERS

    table = jnp.pad(weight.T, ((0, PAD - DIM), (0, 0))).T.reshape(2 * v, DIM)
    idx2 = token_ids * 2

    mesh = plsc.VectorSubcoreMesh(core_axis_name="c", subcore_axis_name="s")
    run = pl.kernel(
        functools.partial(_emb_body, rows_per_w, s),
        out_type=jax.ShapeDtypeStruct((b * s, PAD), jnp.float32),
        mesh=mesh,
        scratch_types=[
            pltpu.VMEM((rows_per_w, s), jnp.int32),
            pltpu.VMEM((NBUF, s, DIM), jnp.float32),
            pltpu.SemaphoreType.DMA,
            pltpu.SemaphoreType.DMA,
        ],
        compiler_params=pltpu.CompilerParams(use_tc_tiling_on_sc=False),
    )
    out = run(idx2, table)
    return out[:, :DIM].reshape(b, s, DIM)


# speedup vs baseline: 1.0934x; 1.0934x over previous
import functools

import jax
import jax.numpy as jnp
from jax import lax
from jax.experimental import pallas as pl
from jax.experimental.pallas import tpu as pltpu
from jax.experimental.pallas import tpu_sc as plsc

DIM = 64
PAD = 128
NUM_WORKERS = 32
NBUF = 4


def _emb_body(rows_per_w, seq, idx_hbm, table_hbm, out_hbm, idx_v, rows_v,
              sem_in, sem_out):
    cid = lax.axis_index("c")
    sid = lax.axis_index("s")
    wid = sid * 2 + cid
    base = wid * rows_per_w
    pltpu.sync_copy(idx_hbm.at[pl.ds(base, rows_per_w)], idx_v)

    def fire_gather(r, buf):
        pltpu.async_copy(table_hbm.at[idx_v.at[r]], rows_v.at[buf], sem_in)

    def wait_one_gather():
        pltpu.make_async_copy(
            table_hbm.at[idx_v.at[0]], rows_v.at[0], sem_in).wait()

    def store(r, buf):
        pltpu.async_copy(
            rows_v.at[buf],
            out_hbm.at[pl.ds((base + r) * seq, seq), pl.ds(0, DIM)], sem_out)

    def wait_one_store():
        pltpu.make_async_copy(
            rows_v.at[0],
            out_hbm.at[pl.ds(0, seq), pl.ds(0, DIM)], sem_out).wait()

    for k in range(NBUF - 1):
        fire_gather(k, k)

    def body(r, carry):
        buf = lax.rem(r, NBUF)

        @pl.when(jnp.logical_and(r >= 1, r + NBUF - 1 < rows_per_w))
        def _():
            wait_one_store()

        @pl.when(r + NBUF - 1 < rows_per_w)
        def _():
            fire_gather(r + NBUF - 1, lax.rem(r + NBUF - 1, NBUF))

        wait_one_gather()
        store(r, buf)
        return carry

    lax.fori_loop(0, rows_per_w, body, 0)
    for _k in range(NBUF):
        wait_one_store()


def kernel(token_ids, weight):
    b, s = token_ids.shape
    v, d = weight.shape
    assert b % NUM_WORKERS == 0 and d == DIM
    rows_per_w = b // NUM_WORKERS

    table = jnp.pad(weight, ((0, 0), (0, PAD - DIM))).reshape(2 * v, DIM)
    idx2 = token_ids * 2

    mesh = plsc.VectorSubcoreMesh(core_axis_name="c", subcore_axis_name="s")
    run = pl.kernel(
        functools.partial(_emb_body, rows_per_w, s),
        out_type=jax.ShapeDtypeStruct((b * s, PAD), jnp.float32),
        mesh=mesh,
        scratch_types=[
            pltpu.VMEM((rows_per_w, s), jnp.int32),
            pltpu.VMEM((NBUF, s, DIM), jnp.float32),
            pltpu.SemaphoreType.DMA,
            pltpu.SemaphoreType.DMA,
        ],
        compiler_params=pltpu.CompilerParams(use_tc_tiling_on_sc=False),
    )
    out = run(idx2, table)
    return out[:, :DIM].reshape(b, s, DIM)
